# baseline (speedup 1.0000x reference)
import functools

import jax
import jax.numpy as jnp
from jax import lax
from jax.experimental import pallas as pl
from jax.experimental.pallas import tpu as pltpu
from jax.experimental.pallas import tpu_sc as plsc

EMBED_DIM = 32
NUM_CORES = 2
NUM_SUBCORES = 16
NUM_WORKERS = NUM_CORES * NUM_SUBCORES
NBUF = 2
SPLIT_B = 4


@jax.jit
def _gather(idx_t, table):
    S, H, B = idx_t.shape
    n_pairs = S * H
    pairs_per_w = n_pairs // NUM_WORKERS
    chunk = B // SPLIT_B
    n_chunks = pairs_per_w * SPLIT_B
    mesh = plsc.VectorSubcoreMesh(core_axis_name="c", subcore_axis_name="s")

    @functools.partial(
        pl.kernel,
        mesh=mesh,
        out_type=jax.ShapeDtypeStruct((B, S, H, EMBED_DIM), jnp.float32),
        scratch_types=[
            pltpu.VMEM((NBUF, chunk), jnp.int32),
            pltpu.VMEM((NBUF, chunk, EMBED_DIM), jnp.float32),
        ]
        + [pltpu.SemaphoreType.DMA] * (3 * NBUF),
        compiler_params=pltpu.CompilerParams(use_tc_tiling_on_sc=False),
    )
    def k(idx_hbm, table_hbm, out_hbm, idx_v, rows_v, *sems):
        isem = sems[0:NBUF]
        gsem = sems[NBUF:2 * NBUF]
        osem = sems[2 * NBUF:3 * NBUF]
        wid = lax.axis_index("s") * NUM_CORES + lax.axis_index("c")
        pair0 = wid * pairs_per_w

        def chunk_coords(c):
            p = pair0 + c // SPLIT_B
            b0 = (c % SPLIT_B) * chunk
            return p // H, p % H, b0

        idx_h, g_h, o_h = {}, {}, {}

        def idx_start(c):
            b = c % NBUF
            s, h, b0 = chunk_coords(c)
            idx_h[c] = pltpu.async_copy(
                idx_hbm.at[s, h, pl.ds(b0, chunk)], idx_v.at[b], isem[b])

        def gather_start(c):
            b = c % NBUF
            g_h[c] = pltpu.async_copy(
                table_hbm.at[idx_v.at[b]], rows_v.at[b], gsem[b])

        def out_start(c):
            b = c % NBUF
            s, h, b0 = chunk_coords(c)
            o_h[c] = pltpu.async_copy(
                rows_v.at[b], out_hbm.at[pl.ds(b0, chunk), s, h], osem[b])

        for c in range(min(NBUF, n_chunks)):
            idx_start(c)
        for c in range(n_chunks):
            if c >= NBUF:
                o_h[c - NBUF].wait()
            idx_h[c].wait()
            gather_start(c)
            g_h[c].wait()
            if c + NBUF < n_chunks:
                idx_start(c + NBUF)
            out_start(c)
        for c in range(max(0, n_chunks - NBUF), n_chunks):
            o_h[c].wait()

    return k(idx_t, table)


def kernel(indices, table):
    idx_t = jnp.transpose(indices, (1, 2, 0))
    return _gather(idx_t, table)
